# baseline (speedup 1.0000x reference)
import functools

import jax
import jax.numpy as jnp
from jax import lax
from jax.experimental import pallas as pl
from jax.experimental.pallas import tpu as pltpu
from jax.experimental.pallas import tpu_sc as plsc

EMB = 64
L = 50
G = 80
SC_BAGS = 8
SC_IDX = SC_BAGS * L
SC_GATHERS = SC_IDX // G


NUM_SC_CORES = 2
NUM_SC_SUBCORES = 16


def _bag_sum(idx2d, table, batch, k_pair):
    nw = NUM_SC_CORES * NUM_SC_SUBCORES
    bags_per_w = batch // nw
    schunks = bags_per_w // SC_BAGS
    idx_rows_per_chunk = SC_IDX // G

    idx_rows_per_w = schunks * idx_rows_per_chunk

    @functools.partial(
        pl.kernel,
        out_type=jax.ShapeDtypeStruct((batch, EMB), jnp.float32),
        mesh=plsc.VectorSubcoreMesh(core_axis_name="c", subcore_axis_name="s",
                                    num_cores=NUM_SC_CORES,
                                    num_subcores=NUM_SC_SUBCORES),
        compiler_params=pltpu.CompilerParams(use_tc_tiling_on_sc=False,
                                             needs_layout_passes=False),
        scratch_types=[
            pltpu.VMEM((idx_rows_per_w, G), jnp.int32),
            pltpu.VMEM((SC_IDX, EMB), jnp.bfloat16),
            pltpu.VMEM((SC_IDX, EMB), jnp.bfloat16),
            pltpu.VMEM((SC_BAGS, EMB), jnp.float32),
            pltpu.VMEM((SC_BAGS, EMB), jnp.float32),
            pltpu.SemaphoreType.DMA,
            pltpu.SemaphoreType.DMA,
            pltpu.SemaphoreType.DMA,
            pltpu.SemaphoreType.DMA,
        ],
    )
    def bag_kernel(idx_hbm, table_hbm, out_hbm, idx_v, rows0, rows1,
                   outv0, outv1, gsem0, gsem1, osem0, osem1):
        wid = lax.axis_index("s") * NUM_SC_CORES + lax.axis_index("c")
        pltpu.sync_copy(idx_hbm.at[pl.ds(wid * idx_rows_per_w, idx_rows_per_w)],
                        idx_v)

        def xform(r, _):
            for c in range(G // 16):
                v = idx_v[r, pl.ds(c * 16, 16)]
                p = v + v - jnp.where(v >= k_pair, 2 * k_pair - 1, 0)
                idx_v[r, pl.ds(c * 16, 16)] = p
            return 0

        lax.fori_loop(0, idx_rows_per_w, xform, 0)

        def fire(chunk, rows_buf, sem):
            for j in range(SC_GATHERS):
                pltpu.async_copy(
                    table_hbm.at[idx_v.at[chunk * idx_rows_per_chunk + j]],
                    rows_buf.at[pl.ds(j * G, G)], sem)

        def drain(chunk, rows_buf, sem):
            for j in range(SC_GATHERS):
                pltpu.make_async_copy(
                    table_hbm.at[idx_v.at[chunk * idx_rows_per_chunk + j]],
                    rows_buf.at[pl.ds(j * G, G)], sem).wait()

        def accumulate(rows_buf, out_buf):
            for b in range(SC_BAGS):
                def body(r, accs, _b=b):
                    base = _b * L + r * 10
                    parts = [[], [], [], []]
                    for k in range(10):
                        x0 = rows_buf[base + k, pl.ds(0, 32)]
                        x1 = rows_buf[base + k, pl.ds(32, 32)]
                        e0, o0 = plsc.unpack(
                            x0, format=plsc.PackFormat.INTERLEAVED)
                        e1, o1 = plsc.unpack(
                            x1, format=plsc.PackFormat.INTERLEAVED)
                        for c, v in enumerate((e0, o0, e1, o1)):
                            parts[c].append(v)
                    new = []
                    for c in range(4):
                        xs = parts[c]
                        t = (((xs[0] + xs[1]) + (xs[2] + xs[3]))
                             + ((xs[4] + xs[5]) + (xs[6] + xs[7]))
                             + (xs[8] + xs[9]))
                        new.append(accs[c] + t)
                    return tuple(new)
                z = jnp.zeros((16,), jnp.float32)
                accs = lax.fori_loop(0, L // 10, body, (z, z, z, z))
                for c in range(4):
                    out_buf[b, pl.ds(c * 16, 16)] = accs[c]

        def out_copy(chunk, out_buf, sem):
            pltpu.async_copy(
                out_buf,
                out_hbm.at[pl.ds(wid * bags_per_w + chunk * SC_BAGS, SC_BAGS)],
                sem)

        def out_drain(chunk, out_buf, sem):
            pltpu.make_async_copy(
                out_buf,
                out_hbm.at[pl.ds(wid * bags_per_w + chunk * SC_BAGS, SC_BAGS)],
                sem).wait()

        fire(0, rows0, gsem0)

        def step(i, _):
            ca = 2 * i
            cb = 2 * i + 1
            drain(ca, rows0, gsem0)
            fire(cb, rows1, gsem1)

            @pl.when(i > 0)
            def _():
                out_drain(ca - 2, outv0, osem0)
            accumulate(rows0, outv0)
            out_copy(ca, outv0, osem0)

            drain(cb, rows1, gsem1)

            @pl.when(cb + 1 < schunks)
            def _():
                fire(cb + 1, rows0, gsem0)

            @pl.when(i > 0)
            def _():
                out_drain(cb - 2, outv1, osem1)
            accumulate(rows1, outv1)
            out_copy(cb, outv1, osem1)
            return 0

        lax.fori_loop(0, schunks // 2, step, 0)
        out_drain(schunks - 2, outv0, osem0)
        out_drain(schunks - 1, outv1, osem1)

    return bag_kernel(idx2d, table)


PACK_BK = 4096


def _pack_nb(vocab):
    return pl.cdiv(pl.cdiv(vocab, 2), PACK_BK)


def _pack_body(lo_ref, hi_ref, out_ref):
    lo = jnp.swapaxes(lo_ref[...], 0, 1)
    hi = jnp.swapaxes(hi_ref[...], 0, 1)
    out_ref[...] = jnp.concatenate([lo, hi], axis=1).astype(jnp.bfloat16)


def _pack(tableT):
    vocab = tableT.shape[1]
    nb = _pack_nb(vocab)
    k_pair = nb * PACK_BK
    last_blk = pl.cdiv(vocab, PACK_BK) - 1
    return pl.pallas_call(
        _pack_body,
        grid=(nb,),
        in_specs=[
            pl.BlockSpec((EMB, PACK_BK), lambda i: (0, i)),
            pl.BlockSpec(
                (EMB, PACK_BK),
                lambda i, _nb=nb, _lb=last_blk: (0, jnp.minimum(i + _nb, _lb))),
        ],
        out_specs=pl.BlockSpec((PACK_BK, 2 * EMB), lambda i: (i, 0)),
        out_shape=jax.ShapeDtypeStruct((k_pair, 2 * EMB), jnp.bfloat16),
    )(tableT, tableT)


def _mlp_body(bag_ref, img_ref, w1a_ref, w1b_ref, b1_ref, w2_ref, b2_ref,
              out_ref):
    h = (jnp.dot(bag_ref[...], w1a_ref[...],
                 preferred_element_type=jnp.float32,
                 precision=lax.Precision.HIGHEST)
         + jnp.dot(img_ref[...], w1b_ref[...],
                   preferred_element_type=jnp.float32,
                   precision=lax.Precision.HIGHEST)
         + b1_ref[...])
    alpha = 1.6732632423543772
    scale = 1.0507009873554805
    h = scale * jnp.where(h > 0, h, alpha * (jnp.exp(jnp.minimum(h, 0.0)) - 1.0))
    y = jnp.sum(h * w2_ref[...], axis=1, keepdims=True) + b2_ref[...]
    out_ref[...] = 1.0 / (1.0 + jnp.exp(-y))


def _mlp(bags, img, w1a, w1b, b1r, w2r, b2r, block=2048):
    batch = bags.shape[0]
    emb = bags.shape[1]
    img_d = img.shape[1]
    hid = w1a.shape[1]
    grid = (batch // block,)
    return pl.pallas_call(
        _mlp_body,
        grid=grid,
        in_specs=[
            pl.BlockSpec((block, emb), lambda i: (i, 0)),
            pl.BlockSpec((block, img_d), lambda i: (i, 0)),
            pl.BlockSpec((emb, hid), lambda i: (0, 0)),
            pl.BlockSpec((img_d, hid), lambda i: (0, 0)),
            pl.BlockSpec((1, hid), lambda i: (0, 0)),
            pl.BlockSpec((1, hid), lambda i: (0, 0)),
            pl.BlockSpec((1, 1), lambda i: (0, 0)),
        ],
        out_specs=pl.BlockSpec((block, 1), lambda i: (i, 0)),
        out_shape=jax.ShapeDtypeStruct((batch, 1), jnp.float32),
    )(bags, img, w1a, w1b, b1r, w2r, b2r)


def kernel(input_text, input_img_feat, batch_size, table, W1, b1, W2, b2):
    batch, seq = input_text.shape
    idx2d = input_text.reshape(batch * seq // G, G)
    k_pair = _pack_nb(table.shape[0]) * PACK_BK
    table_lin = _pack(table.T).reshape(2 * k_pair, EMB)
    bags = _bag_sum(idx2d, table_lin, batch, k_pair)
    perm = ([2 * j for j in range(16)] + [2 * j + 1 for j in range(16)]
            + [32 + 2 * j for j in range(16)] + [32 + 2 * j + 1
                                                 for j in range(16)])
    w1a = W1[:EMB][jnp.array(perm, dtype=jnp.int32)]
    w1b = W1[EMB:]
    return _mlp(bags, input_img_feat, w1a, w1b,
                b1.reshape(1, -1), W2.reshape(1, -1), b2.reshape(1, 1))


# speedup vs baseline: 1.7803x; 1.7803x over previous
import functools

import jax
import jax.numpy as jnp
from jax import lax
from jax.experimental import pallas as pl
from jax.experimental.pallas import tpu as pltpu
from jax.experimental.pallas import tpu_sc as plsc

EMB = 64
L = 50
G = 80
SC_BAGS = 8
SC_IDX = SC_BAGS * L
SC_GATHERS = SC_IDX // G


NUM_SC_CORES = 2
NUM_SC_SUBCORES = 16


def _bag_sum(idx2d, table, batch, k4):
    nw = NUM_SC_CORES * NUM_SC_SUBCORES
    bags_per_w = batch // nw
    schunks = bags_per_w // SC_BAGS
    idx_rows_per_chunk = SC_IDX // G

    idx_rows_per_w = schunks * idx_rows_per_chunk

    @functools.partial(
        pl.kernel,
        out_type=jax.ShapeDtypeStruct((batch, EMB), jnp.float32),
        mesh=plsc.VectorSubcoreMesh(core_axis_name="c", subcore_axis_name="s",
                                    num_cores=NUM_SC_CORES,
                                    num_subcores=NUM_SC_SUBCORES),
        compiler_params=pltpu.CompilerParams(use_tc_tiling_on_sc=False,
                                             needs_layout_passes=False),
        scratch_types=[
            pltpu.VMEM((idx_rows_per_w, G), jnp.int32),
            pltpu.VMEM((SC_IDX, EMB // 2), jnp.float32),
            pltpu.VMEM((SC_IDX, EMB // 2), jnp.float32),
            pltpu.VMEM((SC_BAGS, EMB), jnp.float32),
            pltpu.VMEM((SC_BAGS, EMB), jnp.float32),
            pltpu.SemaphoreType.DMA,
            pltpu.SemaphoreType.DMA,
            pltpu.SemaphoreType.DMA,
            pltpu.SemaphoreType.DMA,
        ],
    )
    def bag_kernel(idx_hbm, table_hbm, out_hbm, idx_v, rows0, rows1,
                   outv0, outv1, gsem0, gsem1, osem0, osem1):
        wid = lax.axis_index("s") * NUM_SC_CORES + lax.axis_index("c")
        pltpu.sync_copy(idx_hbm.at[pl.ds(wid * idx_rows_per_w, idx_rows_per_w)],
                        idx_v)

        def xform(r, _):
            for c in range(G // 16):
                v = idx_v[r, pl.ds(c * 16, 16)]
                s = ((v >= k4).astype(jnp.int32)
                     + (v >= 2 * k4).astype(jnp.int32)
                     + (v >= 3 * k4).astype(jnp.int32))
                idx_v[r, pl.ds(c * 16, 16)] = 4 * v - (4 * k4 - 1) * s
            return 0

        lax.fori_loop(0, idx_rows_per_w, xform, 0)

        def fire(chunk, rows_buf, sem):
            for j in range(SC_GATHERS):
                pltpu.async_copy(
                    table_hbm.at[idx_v.at[chunk * idx_rows_per_chunk + j]],
                    rows_buf.at[pl.ds(j * G, G)], sem)

        def drain(chunk, rows_buf, sem):
            for j in range(SC_GATHERS):
                pltpu.make_async_copy(
                    table_hbm.at[idx_v.at[chunk * idx_rows_per_chunk + j]],
                    rows_buf.at[pl.ds(j * G, G)], sem).wait()

        def accumulate(rows_buf, out_buf):
            for b in range(SC_BAGS):
                def body(r, accs, _b=b):
                    base = _b * L + r * 10
                    parts = [[], [], [], []]
                    for k in range(10):
                        w0 = rows_buf[base + k, pl.ds(0, 16)]
                        w1 = rows_buf[base + k, pl.ds(16, 16)]
                        b0 = plsc.bitcast(w0, jnp.bfloat16)
                        b1 = plsc.bitcast(w1, jnp.bfloat16)
                        e0, o0 = plsc.unpack(
                            b0, format=plsc.PackFormat.INTERLEAVED)
                        e1, o1 = plsc.unpack(
                            b1, format=plsc.PackFormat.INTERLEAVED)
                        for c, v in enumerate((e0, o0, e1, o1)):
                            parts[c].append(v)
                    new = []
                    for c in range(4):
                        xs = parts[c]
                        t = (((xs[0] + xs[1]) + (xs[2] + xs[3]))
                             + ((xs[4] + xs[5]) + (xs[6] + xs[7]))
                             + (xs[8] + xs[9]))
                        new.append(accs[c] + t)
                    return tuple(new)
                z = jnp.zeros((16,), jnp.float32)
                accs = lax.fori_loop(0, L // 10, body, (z, z, z, z))
                for c in range(4):
                    out_buf[b, pl.ds(c * 16, 16)] = accs[c]

        def out_copy(chunk, out_buf, sem):
            pltpu.async_copy(
                out_buf,
                out_hbm.at[pl.ds(wid * bags_per_w + chunk * SC_BAGS, SC_BAGS)],
                sem)

        def out_drain(chunk, out_buf, sem):
            pltpu.make_async_copy(
                out_buf,
                out_hbm.at[pl.ds(wid * bags_per_w + chunk * SC_BAGS, SC_BAGS)],
                sem).wait()

        fire(0, rows0, gsem0)

        def step(i, _):
            ca = 2 * i
            cb = 2 * i + 1
            drain(ca, rows0, gsem0)
            fire(cb, rows1, gsem1)

            @pl.when(i > 0)
            def _():
                out_drain(ca - 2, outv0, osem0)
            accumulate(rows0, outv0)
            out_copy(ca, outv0, osem0)

            drain(cb, rows1, gsem1)

            @pl.when(cb + 1 < schunks)
            def _():
                fire(cb + 1, rows0, gsem0)

            @pl.when(i > 0)
            def _():
                out_drain(cb - 2, outv1, osem1)
            accumulate(rows1, outv1)
            out_copy(cb, outv1, osem1)
            return 0

        lax.fori_loop(0, schunks // 2, step, 0)
        out_drain(schunks - 2, outv0, osem0)
        out_drain(schunks - 1, outv1, osem1)

    return bag_kernel(idx2d, table)


PACK_BK = 4096


def _pack_nb(vocab):
    return pl.cdiv(pl.cdiv(vocab, 4), PACK_BK)


def _pack_body(q0_ref, q1_ref, q2_ref, q3_ref, out_ref):
    groups = []
    for ref in (q0_ref, q1_ref, q2_ref, q3_ref):
        x = jnp.swapaxes(ref[...], 0, 1).astype(jnp.bfloat16)
        lo = lax.bitcast_convert_type(x[:, :EMB // 2],
                                      jnp.uint16).astype(jnp.uint32)
        hi = lax.bitcast_convert_type(x[:, EMB // 2:],
                                      jnp.uint16).astype(jnp.uint32)
        w = lo | (hi << 16)
        groups.append(lax.bitcast_convert_type(w, jnp.float32))
    out_ref[...] = jnp.concatenate(groups, axis=1)


def _pack(tableT):
    vocab = tableT.shape[1]
    nb = _pack_nb(vocab)
    k4 = nb * PACK_BK
    last_blk = pl.cdiv(vocab, PACK_BK) - 1

    def make_spec(off):
        return pl.BlockSpec(
            (EMB, PACK_BK),
            lambda i, _o=off, _lb=last_blk: (0, jnp.minimum(i + _o, _lb)))

    return pl.pallas_call(
        _pack_body,
        grid=(nb,),
        in_specs=[make_spec(q * nb) for q in range(4)],
        out_specs=pl.BlockSpec((PACK_BK, 2 * EMB), lambda i: (i, 0)),
        out_shape=jax.ShapeDtypeStruct((k4, 2 * EMB), jnp.float32),
    )(tableT, tableT, tableT, tableT)


def _mlp_body(bag_ref, img_ref, w1a_ref, w1b_ref, b1_ref, w2_ref, b2_ref,
              out_ref):
    h = (jnp.dot(bag_ref[...], w1a_ref[...],
                 preferred_element_type=jnp.float32,
                 precision=lax.Precision.HIGHEST)
         + jnp.dot(img_ref[...], w1b_ref[...],
                   preferred_element_type=jnp.float32,
                   precision=lax.Precision.HIGHEST)
         + b1_ref[...])
    alpha = 1.6732632423543772
    scale = 1.0507009873554805
    h = scale * jnp.where(h > 0, h, alpha * (jnp.exp(jnp.minimum(h, 0.0)) - 1.0))
    y = jnp.sum(h * w2_ref[...], axis=1, keepdims=True) + b2_ref[...]
    out_ref[...] = 1.0 / (1.0 + jnp.exp(-y))


def _mlp(bags, img, w1a, w1b, b1r, w2r, b2r, block=2048):
    batch = bags.shape[0]
    emb = bags.shape[1]
    img_d = img.shape[1]
    hid = w1a.shape[1]
    grid = (batch // block,)
    return pl.pallas_call(
        _mlp_body,
        grid=grid,
        in_specs=[
            pl.BlockSpec((block, emb), lambda i: (i, 0)),
            pl.BlockSpec((block, img_d), lambda i: (i, 0)),
            pl.BlockSpec((emb, hid), lambda i: (0, 0)),
            pl.BlockSpec((img_d, hid), lambda i: (0, 0)),
            pl.BlockSpec((1, hid), lambda i: (0, 0)),
            pl.BlockSpec((1, hid), lambda i: (0, 0)),
            pl.BlockSpec((1, 1), lambda i: (0, 0)),
        ],
        out_specs=pl.BlockSpec((block, 1), lambda i: (i, 0)),
        out_shape=jax.ShapeDtypeStruct((batch, 1), jnp.float32),
    )(bags, img, w1a, w1b, b1r, w2r, b2r)


def kernel(input_text, input_img_feat, batch_size, table, W1, b1, W2, b2):
    batch, seq = input_text.shape
    idx2d = input_text.reshape(batch * seq // G, G)
    k4 = _pack_nb(table.shape[0]) * PACK_BK
    table_words = _pack(table.T).reshape(4 * k4, EMB // 2)
    bags = _bag_sum(idx2d, table_words, batch, k4)
    perm = [(0, 32, 16, 48)[j // 16] + j % 16 for j in range(EMB)]
    w1a = W1[:EMB][jnp.array(perm, dtype=jnp.int32)]
    w1b = W1[EMB:]
    return _mlp(bags, input_img_feat, w1a, w1b,
                b1.reshape(1, -1), W2.reshape(1, -1), b2.reshape(1, 1))


# speedup vs baseline: 1.9423x; 1.0910x over previous
import functools

import jax
import jax.numpy as jnp
from jax import lax
from jax.experimental import pallas as pl
from jax.experimental.pallas import tpu as pltpu
from jax.experimental.pallas import tpu_sc as plsc

EMB = 64
L = 50
G = 80
SC_BAGS = 8
SC_IDX = SC_BAGS * L
SC_GATHERS = SC_IDX // G


NUM_SC_CORES = 2
NUM_SC_SUBCORES = 16


def _bag_sum(idx2d, table, batch, k4):
    nw = NUM_SC_CORES * NUM_SC_SUBCORES
    bags_per_w = batch // nw
    schunks = bags_per_w // SC_BAGS
    idx_rows_per_chunk = SC_IDX // G

    idx_rows_per_w = schunks * idx_rows_per_chunk

    @functools.partial(
        pl.kernel,
        out_type=jax.ShapeDtypeStruct((batch, EMB), jnp.float32),
        mesh=plsc.VectorSubcoreMesh(core_axis_name="c", subcore_axis_name="s",
                                    num_cores=NUM_SC_CORES,
                                    num_subcores=NUM_SC_SUBCORES),
        compiler_params=pltpu.CompilerParams(use_tc_tiling_on_sc=False,
                                             needs_layout_passes=False),
        scratch_types=[
            pltpu.VMEM((idx_rows_per_w, G), jnp.int32),
            pltpu.VMEM((SC_IDX, EMB // 2), jnp.float32),
            pltpu.VMEM((SC_IDX, EMB // 2), jnp.float32),
            pltpu.VMEM((SC_BAGS, EMB), jnp.float32),
            pltpu.VMEM((SC_BAGS, EMB), jnp.float32),
            pltpu.SemaphoreType.DMA,
            pltpu.SemaphoreType.DMA,
            pltpu.SemaphoreType.DMA,
            pltpu.SemaphoreType.DMA,
        ],
    )
    def bag_kernel(idx_hbm, table_hbm, out_hbm, idx_v, rows0, rows1,
                   outv0, outv1, gsem0, gsem1, osem0, osem1):
        wid = lax.axis_index("s") * NUM_SC_CORES + lax.axis_index("c")
        pltpu.sync_copy(idx_hbm.at[pl.ds(wid * idx_rows_per_w, idx_rows_per_w)],
                        idx_v)

        def xform(r, _):
            for c in range(G // 16):
                v = idx_v[r, pl.ds(c * 16, 16)]
                s = ((v >= k4).astype(jnp.int32)
                     + (v >= 2 * k4).astype(jnp.int32)
                     + (v >= 3 * k4).astype(jnp.int32))
                idx_v[r, pl.ds(c * 16, 16)] = 4 * v - (4 * k4 - 1) * s
            return 0

        lax.fori_loop(0, idx_rows_per_w, xform, 0)

        def fire(chunk, rows_buf, sem):
            for j in range(SC_GATHERS):
                pltpu.async_copy(
                    table_hbm.at[idx_v.at[chunk * idx_rows_per_chunk + j]],
                    rows_buf.at[pl.ds(j * G, G)], sem)

        def drain(chunk, rows_buf, sem):
            for j in range(SC_GATHERS):
                pltpu.make_async_copy(
                    table_hbm.at[idx_v.at[chunk * idx_rows_per_chunk + j]],
                    rows_buf.at[pl.ds(j * G, G)], sem).wait()

        def accumulate(rows_buf, out_buf):
            for b in range(SC_BAGS):
                def body(r, accs, _b=b):
                    base = _b * L + r * 10
                    parts = [[], [], [], []]
                    for k in range(10):
                        w0 = rows_buf[base + k, pl.ds(0, 16)]
                        w1 = rows_buf[base + k, pl.ds(16, 16)]
                        b0 = plsc.bitcast(w0, jnp.bfloat16)
                        b1 = plsc.bitcast(w1, jnp.bfloat16)
                        e0, o0 = plsc.unpack(
                            b0, format=plsc.PackFormat.INTERLEAVED)
                        e1, o1 = plsc.unpack(
                            b1, format=plsc.PackFormat.INTERLEAVED)
                        for c, v in enumerate((e0, o0, e1, o1)):
                            parts[c].append(v)
                    new = []
                    for c in range(4):
                        xs = parts[c]
                        t = (((xs[0] + xs[1]) + (xs[2] + xs[3]))
                             + ((xs[4] + xs[5]) + (xs[6] + xs[7]))
                             + (xs[8] + xs[9]))
                        new.append(accs[c] + t)
                    return tuple(new)
                z = jnp.zeros((16,), jnp.float32)
                accs = lax.fori_loop(0, L // 10, body, (z, z, z, z))
                for c in range(4):
                    out_buf[b, pl.ds(c * 16, 16)] = accs[c]

        def out_copy(chunk, out_buf, sem):
            pltpu.async_copy(
                out_buf,
                out_hbm.at[pl.ds(wid * bags_per_w + chunk * SC_BAGS, SC_BAGS)],
                sem)

        def out_drain(chunk, out_buf, sem):
            pltpu.make_async_copy(
                out_buf,
                out_hbm.at[pl.ds(wid * bags_per_w + chunk * SC_BAGS, SC_BAGS)],
                sem).wait()

        fire(0, rows0, gsem0)

        def step(i, _):
            ca = 2 * i
            cb = 2 * i + 1
            drain(ca, rows0, gsem0)
            fire(cb, rows1, gsem1)

            @pl.when(i > 0)
            def _():
                out_drain(ca - 2, outv0, osem0)
            accumulate(rows0, outv0)
            out_copy(ca, outv0, osem0)

            drain(cb, rows1, gsem1)

            @pl.when(cb + 1 < schunks)
            def _():
                fire(cb + 1, rows0, gsem0)

            @pl.when(i > 0)
            def _():
                out_drain(cb - 2, outv1, osem1)
            accumulate(rows1, outv1)
            out_copy(cb, outv1, osem1)
            return 0

        lax.fori_loop(0, schunks // 2, step, 0)
        out_drain(schunks - 2, outv0, osem0)
        out_drain(schunks - 1, outv1, osem1)

    return bag_kernel(idx2d, table)


PACK_BK = 4096


def _pack_nb(vocab):
    return pl.cdiv(pl.cdiv(vocab, 4), PACK_BK)


def _pack_body(q0_ref, q1_ref, q2_ref, q3_ref, out_ref):
    groups = []
    for ref in (q0_ref, q1_ref, q2_ref, q3_ref):
        x = jnp.swapaxes(ref[...].astype(jnp.bfloat16), 0, 1)
        lo = lax.bitcast_convert_type(x[:, :EMB // 2],
                                      jnp.uint16).astype(jnp.uint32)
        hi = lax.bitcast_convert_type(x[:, EMB // 2:],
                                      jnp.uint16).astype(jnp.uint32)
        w = lo | (hi << 16)
        groups.append(lax.bitcast_convert_type(w, jnp.float32))
    out_ref[...] = jnp.concatenate(groups, axis=1)


def _pack(tableT):
    vocab = tableT.shape[1]
    nb = _pack_nb(vocab)
    k4 = nb * PACK_BK
    last_blk = pl.cdiv(vocab, PACK_BK) - 1

    def make_spec(off):
        return pl.BlockSpec(
            (EMB, PACK_BK),
            lambda i, _o=off, _lb=last_blk: (0, jnp.minimum(i + _o, _lb)))

    return pl.pallas_call(
        _pack_body,
        grid=(nb,),
        in_specs=[make_spec(q * nb) for q in range(4)],
        out_specs=pl.BlockSpec((PACK_BK, 2 * EMB), lambda i: (i, 0)),
        out_shape=jax.ShapeDtypeStruct((k4, 2 * EMB), jnp.float32),
    )(tableT, tableT, tableT, tableT)


def _mlp_body(bag_ref, img_ref, w1a_ref, w1b_ref, b1_ref, w2_ref, b2_ref,
              out_ref):
    h = (jnp.dot(bag_ref[...], w1a_ref[...],
                 preferred_element_type=jnp.float32)
         + jnp.dot(img_ref[...], w1b_ref[...],
                   preferred_element_type=jnp.float32)
         + b1_ref[...])
    alpha = 1.6732632423543772
    scale = 1.0507009873554805
    h = scale * jnp.where(h > 0, h, alpha * (jnp.exp(jnp.minimum(h, 0.0)) - 1.0))
    y = jnp.sum(h * w2_ref[...], axis=1, keepdims=True) + b2_ref[...]
    out_ref[...] = 1.0 / (1.0 + jnp.exp(-y))


def _mlp(bags, img, w1a, w1b, b1r, w2r, b2r, block=2048):
    batch = bags.shape[0]
    emb = bags.shape[1]
    img_d = img.shape[1]
    hid = w1a.shape[1]
    grid = (batch // block,)
    return pl.pallas_call(
        _mlp_body,
        grid=grid,
        in_specs=[
            pl.BlockSpec((block, emb), lambda i: (i, 0)),
            pl.BlockSpec((block, img_d), lambda i: (i, 0)),
            pl.BlockSpec((emb, hid), lambda i: (0, 0)),
            pl.BlockSpec((img_d, hid), lambda i: (0, 0)),
            pl.BlockSpec((1, hid), lambda i: (0, 0)),
            pl.BlockSpec((1, hid), lambda i: (0, 0)),
            pl.BlockSpec((1, 1), lambda i: (0, 0)),
        ],
        out_specs=pl.BlockSpec((block, 1), lambda i: (i, 0)),
        out_shape=jax.ShapeDtypeStruct((batch, 1), jnp.float32),
    )(bags, img, w1a, w1b, b1r, w2r, b2r)


def kernel(input_text, input_img_feat, batch_size, table, W1, b1, W2, b2):
    batch, seq = input_text.shape
    idx2d = input_text.reshape(batch * seq // G, G)
    k4 = _pack_nb(table.shape[0]) * PACK_BK
    table_words = _pack(table.T).reshape(4 * k4, EMB // 2)
    bags = _bag_sum(idx2d, table_words, batch, k4)
    perm = [(0, 32, 16, 48)[j // 16] + j % 16 for j in range(EMB)]
    w1a = W1[:EMB][jnp.array(perm, dtype=jnp.int32)]
    w1b = W1[EMB:]
    return _mlp(bags, input_img_feat, w1a, w1b,
                b1.reshape(1, -1), W2.reshape(1, -1), b2.reshape(1, 1))
